# Initial kernel scaffold; baseline (speedup 1.0000x reference)
import jax
import jax.numpy as jnp
from jax.experimental import pallas as pl


def kernel(x, scalars, p, edge_index, W, b):
    raise NotImplementedError("write your pallas kernel here")


# speedup vs baseline: 4.9891x; 4.9891x over previous
import functools

import jax
import jax.numpy as jnp
from jax import lax
from jax.experimental import pallas as pl
from jax.experimental.pallas import tpu as pltpu
from jax.experimental.pallas import tpu_sc as plsc

NC = 2
NS = 16
CHUNK = 128


def _mm_body(x_ref, s_ref, w1_ref, w2_ref, b_ref, h_ref):
    h = jnp.dot(x_ref[...], w1_ref[...], preferred_element_type=jnp.float32)
    h = h + jnp.dot(s_ref[...], w2_ref[...], preferred_element_type=jnp.float32)
    h_ref[...] = h + b_ref[...]


def _matmul(x, s, w1, w2, b, block_rows):
    n, d = x.shape
    d_out = w1.shape[1]
    grid = (n // block_rows,)
    return pl.pallas_call(
        _mm_body,
        grid=grid,
        in_specs=[
            pl.BlockSpec((block_rows, d), lambda i: (i, 0)),
            pl.BlockSpec((1, s.shape[1]), lambda i: (0, 0)),
            pl.BlockSpec((d, d_out), lambda i: (0, 0)),
            pl.BlockSpec((s.shape[1], d_out), lambda i: (0, 0)),
            pl.BlockSpec((1, d_out), lambda i: (0, 0)),
        ],
        out_specs=pl.BlockSpec((block_rows, d_out), lambda i: (i, 0)),
        out_shape=jax.ShapeDtypeStruct((n, d_out), jnp.float32),
    )(x, s, w1, w2, b)


def _add_body(a_ref, b_ref, o_ref):
    o_ref[...] = a_ref[...] + b_ref[...]


def _add(a, b, block_rows):
    n, d = a.shape
    grid = (n // block_rows,)
    spec = pl.BlockSpec((block_rows, d), lambda i: (i, 0))
    return pl.pallas_call(
        _add_body,
        grid=grid,
        in_specs=[spec, spec],
        out_specs=spec,
        out_shape=jax.ShapeDtypeStruct((n, d), jnp.float32),
    )(a, b)


def _make_sc_scatter(n, d_out, ch):
    acc_rows = ((n + 1 + 8 * NS - 1) // (8 * NS)) * (8 * NS)
    zrows = acc_rows // NS
    rpt = (n // NS) // 8 * 8
    tail = n - NS * rpt
    mesh = plsc.VectorSubcoreMesh(
        core_axis_name="c", subcore_axis_name="s", num_cores=NC, num_subcores=NS
    )

    @functools.partial(
        pl.kernel,
        mesh=mesh,
        out_type=[
            jax.ShapeDtypeStruct((n, d_out), jnp.float32),
            jax.ShapeDtypeStruct((n, d_out), jnp.float32),
        ],
        scratch_types=[
            pltpu.VMEM((ch, CHUNK), jnp.int32),
            pltpu.VMEM((ch, CHUNK), jnp.int32),
            pltpu.VMEM((CHUNK, d_out), jnp.float32),
            pltpu.VMEM_SHARED((acc_rows, d_out), jnp.float32),
            pltpu.SemaphoreType.DMA,
        ],
    )
    def sc_kernel(h, src_h, dst_h, o0, o1, srcv, dstv, rows, acc, sem):
        c = lax.axis_index("c")
        s = lax.axis_index("s")
        wid = c * NS + s

        zero16 = jnp.zeros((16,), jnp.float32)

        def zrow(i, carry):
            for j in range(d_out // 16):
                rows[i, pl.ds(j * 16, 16)] = zero16
            return carry

        lax.fori_loop(0, CHUNK, zrow, 0)
        base = s * zrows
        nfull = zrows // CHUNK
        for k in range(nfull):
            pltpu.sync_copy(rows, acc.at[pl.ds(base + k * CHUNK, CHUNK)])
        rem = zrows - nfull * CHUNK
        if rem:
            pltpu.sync_copy(
                rows.at[pl.ds(0, rem)], acc.at[pl.ds(base + nfull * CHUNK, rem)]
            )

        pltpu.sync_copy(src_h.at[wid], srcv)
        pltpu.sync_copy(dst_h.at[wid], dstv)
        plsc.subcore_barrier()

        def body(j, carry):
            pltpu.async_copy(h.at[srcv.at[j]], rows, sem).wait()
            pltpu.sync_copy(rows, acc.at[dstv.at[j]], add=True)
            return carry

        lax.fori_loop(0, ch, body, 0)
        plsc.subcore_barrier()

        @pl.when(c == 0)
        def _():
            pltpu.sync_copy(acc.at[pl.ds(s * rpt, rpt)], o0.at[pl.ds(s * rpt, rpt)])
            if tail:
                @pl.when(s == 0)
                def _():
                    pltpu.sync_copy(
                        acc.at[pl.ds(NS * rpt, tail)], o0.at[pl.ds(NS * rpt, tail)]
                    )

        @pl.when(c == 1)
        def _():
            pltpu.sync_copy(acc.at[pl.ds(s * rpt, rpt)], o1.at[pl.ds(s * rpt, rpt)])
            if tail:
                @pl.when(s == 0)
                def _():
                    pltpu.sync_copy(
                        acc.at[pl.ds(NS * rpt, tail)], o1.at[pl.ds(NS * rpt, tail)]
                    )

    return sc_kernel


def kernel(x, scalars, p, edge_index, W, b):
    n, d = x.shape
    d_out = W.shape[1]
    e = edge_index.shape[1]

    w1 = W[:d]
    w2 = W[d:]
    s2d = scalars.reshape(1, -1).astype(jnp.float32)
    b2d = b.reshape(1, -1)
    h = _matmul(x, s2d, w1, w2, b2d, block_rows=2000)

    nw = NC * NS
    ch = (e + nw * CHUNK - 1) // (nw * CHUNK)
    e_pad = nw * ch * CHUNK
    src = edge_index[0].astype(jnp.int32)
    dst = edge_index[1].astype(jnp.int32)
    pad = e_pad - e
    src = jnp.concatenate([src, jnp.zeros((pad,), jnp.int32)])
    dst = jnp.concatenate([dst, jnp.full((pad,), n, jnp.int32)])
    src3 = src.reshape(nw, ch, CHUNK)
    dst3 = dst.reshape(nw, ch, CHUNK)

    o0, o1 = _make_sc_scatter(n, d_out, ch)(h, src3, dst3)
    return _add(o0, o1, block_rows=2000)
